# baseline (speedup 1.0000x reference)
import functools

import jax
import jax.numpy as jnp
from jax import lax
from jax.experimental import pallas as pl
from jax.experimental.pallas import tpu as pltpu
from jax.experimental.pallas import tpu_sc as plsc

N = 10000
D = 256
DH = 128
E = 160000

N_PAD = 10112
RPT = N_PAD // 16

CHUNK = 120
CPT = 84
E_PAD = CHUNK * 16 * CPT
NBUF = 3
NIB = 4

N_DEG = 10240
RPT_D = N_DEG // 16
CHUNK_D = 1680
CPT_D = (E_PAD // CHUNK_D) // 32

BLK = 1024
GRID = 10

_mesh = plsc.VectorSubcoreMesh(core_axis_name="c", subcore_axis_name="s")


def _deg_body(dst_hbm, pd_hbm, dstv, onesv, zerov, dacc):
    c = lax.axis_index("c")
    s = lax.axis_index("s")
    wid = c * 16 + s
    for i in range(CHUNK_D // 16):
        onesv[pl.ds(i * 16, 16)] = jnp.full((16,), 1.0, jnp.float32)
    for i in range(RPT_D // 16):
        zerov[pl.ds(i * 16, 16)] = jnp.zeros((16,), jnp.float32)
    pltpu.sync_copy(zerov, dacc.at[pl.ds(s * RPT_D, RPT_D)])
    plsc.subcore_barrier()

    def body(j, carry):
        off = (wid * CPT_D + j) * CHUNK_D
        pltpu.sync_copy(dst_hbm.at[pl.ds(off, CHUNK_D)], dstv)
        pltpu.sync_copy(onesv, dacc.at[dstv], add=True)
        return carry

    lax.fori_loop(0, CPT_D, body, 0)
    plsc.subcore_barrier()
    pltpu.sync_copy(dacc.at[pl.ds(s * RPT_D, RPT_D)],
                    pd_hbm.at[c, pl.ds(s * RPT_D, RPT_D)])


_deg_kernel = functools.partial(
    pl.kernel,
    out_type=jax.ShapeDtypeStruct((2, N_DEG), jnp.float32),
    mesh=_mesh,
    scratch_types=[
        pltpu.VMEM((CHUNK_D,), jnp.int32),
        pltpu.VMEM((CHUNK_D,), jnp.float32),
        pltpu.VMEM((RPT_D,), jnp.float32),
        pltpu.VMEM_SHARED((N_DEG,), jnp.float32),
    ],
)(_deg_body)


def _mp_body(src_hbm, dst_hbm, u0_hbm, u1_hbm, z_hbm, s0_hbm, s1_hbm,
             srcv0, srcv1, srcv2, srcv3,
             dstv0, dstv1, dstv2, dstv3,
             rows0, rows1, rows2,
             acc, zsem, isem, gsem, ssem):
    srcv = [srcv0, srcv1, srcv2, srcv3]
    dstv = [dstv0, dstv1, dstv2, dstv3]
    rows = [rows0, rows1, rows2]
    c = lax.axis_index("c")
    s = lax.axis_index("s")
    zcp = pltpu.async_copy(z_hbm.at[pl.ds(s * RPT, RPT)],
                           acc.at[pl.ds(s * RPT, RPT)], zsem)

    idx_cps = {}

    def load_idx(j):
        b = j % NIB
        off = (s * CPT + j) * CHUNK
        d1 = pltpu.async_copy(src_hbm.at[pl.ds(off, CHUNK)], srcv[b], isem)
        d2 = pltpu.async_copy(dst_hbm.at[pl.ds(off, CHUNK)], dstv[b], isem)
        idx_cps[j] = (d1, d2)

    def start_gather(j):
        ib = j % NIB
        b = j % NBUF

        @pl.when(c == 0)
        def _():
            pltpu.async_copy(u0_hbm.at[srcv[ib]], rows[b], gsem)

        @pl.when(c == 1)
        def _():
            pltpu.async_copy(u1_hbm.at[srcv[ib]], rows[b], gsem)

    def wait_gather(j):
        ib = j % NIB
        b = j % NBUF
        pltpu.make_async_copy(u0_hbm.at[srcv[ib]], rows[b], gsem).wait()

    def start_scatter(j):
        ib = j % NIB
        b = j % NBUF
        pltpu.async_copy(rows[b], acc.at[dstv[ib]], ssem, add=True)

    def wait_scatter(j):
        ib = j % NIB
        b = j % NBUF
        pltpu.make_async_copy(rows[b], acc.at[dstv[ib]], ssem).wait()

    load_idx(0)
    load_idx(1)
    idx_cps[0][0].wait()
    idx_cps[0][1].wait()
    start_gather(0)
    zcp.wait()
    plsc.subcore_barrier()

    for j in range(CPT):
        wait_gather(j)
        if j >= 2:
            wait_scatter(j - 2)
        if j + 2 < CPT:
            load_idx(j + 2)
        if j + 1 < CPT:
            idx_cps[j + 1][0].wait()
            idx_cps[j + 1][1].wait()
            start_gather(j + 1)
        start_scatter(j)
    wait_scatter(CPT - 2)
    wait_scatter(CPT - 1)
    plsc.subcore_barrier()

    @pl.when(c == 0)
    def _():
        pltpu.sync_copy(acc.at[pl.ds(s * RPT, RPT)],
                        s0_hbm.at[pl.ds(s * RPT, RPT)])

    @pl.when(c == 1)
    def _():
        pltpu.sync_copy(acc.at[pl.ds(s * RPT, RPT)],
                        s1_hbm.at[pl.ds(s * RPT, RPT)])


_mp_kernel = functools.partial(
    pl.kernel,
    out_type=(jax.ShapeDtypeStruct((N_PAD, DH), jnp.float32),
              jax.ShapeDtypeStruct((N_PAD, DH), jnp.float32)),
    mesh=_mesh,
    scratch_types=(
        [pltpu.VMEM((CHUNK,), jnp.int32)] * 8
        + [pltpu.VMEM((CHUNK, DH), jnp.float32)] * 3
    ) + [
        pltpu.VMEM_SHARED((N_PAD, DH), jnp.float32),
        pltpu.SemaphoreType.DMA,
        pltpu.SemaphoreType.DMA,
        pltpu.SemaphoreType.DMA,
        pltpu.SemaphoreType.DMA,
    ],
)(_mp_body)


def _enc_body(x_ref, dis_ref, We_ref, be_ref, W1_ref, b1_ref,
              h0_ref, u0_ref, u1_ref):
    h0 = jnp.dot(x_ref[...], We_ref[...],
                 preferred_element_type=jnp.float32) + be_ref[...]
    t1 = jnp.dot(h0, W1_ref[...],
                 preferred_element_type=jnp.float32) + b1_ref[...]
    u = t1 * dis_ref[...]
    h0_ref[...] = h0.astype(jnp.bfloat16)
    u0_ref[...] = u[:, :DH]
    u1_ref[...] = u[:, DH:]


_enc_call = pl.pallas_call(
    _enc_body,
    grid=(GRID,),
    in_specs=[
        pl.BlockSpec((BLK, D), lambda i: (i, 0)),
        pl.BlockSpec((BLK, 1), lambda i: (i, 0)),
        pl.BlockSpec((D, D), lambda i: (0, 0)),
        pl.BlockSpec((1, D), lambda i: (0, 0)),
        pl.BlockSpec((D, D), lambda i: (0, 0)),
        pl.BlockSpec((1, D), lambda i: (0, 0)),
    ],
    out_specs=[
        pl.BlockSpec((BLK, D), lambda i: (i, 0)),
        pl.BlockSpec((BLK, DH), lambda i: (i, 0)),
        pl.BlockSpec((BLK, DH), lambda i: (i, 0)),
    ],
    out_shape=[
        jax.ShapeDtypeStruct((N, D), jnp.bfloat16),
        jax.ShapeDtypeStruct((N_PAD, DH), jnp.float32),
        jax.ShapeDtypeStruct((N_PAD, DH), jnp.float32),
    ],
)


def _mid_body(s0_ref, s1_ref, u0_ref, u1_ref, dis_ref, h_ref, W_ref, b_ref,
              hn_ref, v0_ref, v1_ref):
    pre = jnp.concatenate([s0_ref[...] + u0_ref[...],
                           s1_ref[...] + u1_ref[...]], axis=1)
    hn = jnp.maximum(pre * dis_ref[...], 0.0) + h_ref[...].astype(jnp.float32)
    t = jnp.dot(hn, W_ref[...], preferred_element_type=jnp.float32) + b_ref[...]
    u = t * dis_ref[...]
    hn_ref[...] = hn.astype(jnp.bfloat16)
    v0_ref[...] = u[:, :DH]
    v1_ref[...] = u[:, DH:]


_mid_call = pl.pallas_call(
    _mid_body,
    grid=(GRID,),
    in_specs=[
        pl.BlockSpec((BLK, DH), lambda i: (i, 0)),
        pl.BlockSpec((BLK, DH), lambda i: (i, 0)),
        pl.BlockSpec((BLK, DH), lambda i: (i, 0)),
        pl.BlockSpec((BLK, DH), lambda i: (i, 0)),
        pl.BlockSpec((BLK, 1), lambda i: (i, 0)),
        pl.BlockSpec((BLK, D), lambda i: (i, 0)),
        pl.BlockSpec((D, D), lambda i: (0, 0)),
        pl.BlockSpec((1, D), lambda i: (0, 0)),
    ],
    out_specs=[
        pl.BlockSpec((BLK, D), lambda i: (i, 0)),
        pl.BlockSpec((BLK, DH), lambda i: (i, 0)),
        pl.BlockSpec((BLK, DH), lambda i: (i, 0)),
    ],
    out_shape=[
        jax.ShapeDtypeStruct((N, D), jnp.bfloat16),
        jax.ShapeDtypeStruct((N_PAD, DH), jnp.float32),
        jax.ShapeDtypeStruct((N_PAD, DH), jnp.float32),
    ],
)


def _dec_body(s0_ref, s1_ref, u0_ref, u1_ref, dis_ref, h_ref, W_ref, b_ref,
              out_ref):
    pre = jnp.concatenate([s0_ref[...] + u0_ref[...],
                           s1_ref[...] + u1_ref[...]], axis=1)
    hn = jnp.maximum(pre * dis_ref[...], 0.0) + h_ref[...].astype(jnp.float32)
    out_ref[...] = jnp.dot(hn, W_ref[...],
                           preferred_element_type=jnp.float32) + b_ref[...]


_dec_call = pl.pallas_call(
    _dec_body,
    grid=(GRID,),
    in_specs=[
        pl.BlockSpec((BLK, DH), lambda i: (i, 0)),
        pl.BlockSpec((BLK, DH), lambda i: (i, 0)),
        pl.BlockSpec((BLK, DH), lambda i: (i, 0)),
        pl.BlockSpec((BLK, DH), lambda i: (i, 0)),
        pl.BlockSpec((BLK, 1), lambda i: (i, 0)),
        pl.BlockSpec((BLK, D), lambda i: (i, 0)),
        pl.BlockSpec((D, D), lambda i: (0, 0)),
        pl.BlockSpec((1, D), lambda i: (0, 0)),
    ],
    out_specs=pl.BlockSpec((BLK, D), lambda i: (i, 0)),
    out_shape=jax.ShapeDtypeStruct((N, D), jnp.float32),
)


def kernel(x, edge_index, W_enc, b_enc, W_c1, b_c1, W_c2, b_c2, W_dec, b_dec):
    src = edge_index[0]
    dst = edge_index[1]
    pad = jnp.full((E_PAD - E,), N, jnp.int32)
    srcp = jnp.concatenate([src, pad])
    dstp = jnp.concatenate([dst, pad])

    pd = _deg_kernel(dstp)
    deg = pd[0, :N] + pd[1, :N] + 1.0
    disc = lax.rsqrt(deg).reshape(N, 1)

    be = b_enc.reshape(1, D)
    b1 = b_c1.reshape(1, D)
    b2 = b_c2.reshape(1, D)
    bd = b_dec.reshape(1, D)
    zeros_acc = jnp.zeros((N_PAD, DH), jnp.float32)

    h0, u0, u1 = _enc_call(x, disc, W_enc, be, W_c1, b1)
    s0, s1 = _mp_kernel(srcp, dstp, u0, u1, zeros_acc)
    h1, v0, v1 = _mid_call(s0, s1, u0, u1, disc, h0, W_c2, b2)
    r0, r1 = _mp_kernel(srcp, dstp, v0, v1, zeros_acc)
    out = _dec_call(r0, r1, v0, v1, disc, h1, W_dec, bd)
    return out


# speedup vs baseline: 1.0320x; 1.0320x over previous
import functools

import jax
import jax.numpy as jnp
from jax import lax
from jax.experimental import pallas as pl
from jax.experimental.pallas import tpu as pltpu
from jax.experimental.pallas import tpu_sc as plsc

N = 10000
D = 256
DH = 128
E = 160000

N_PAD = 10112
RPT = N_PAD // 16

CHUNK = 120
CPT = 84
E_PAD = CHUNK * 16 * CPT
NBUF = 3
NIB = 4

N_DEG = 10240
RPT_D = N_DEG // 16
CHUNK_D = 1680
CPT_D = (E_PAD // CHUNK_D) // 32

BLK = 1024
GRID = 10

_mesh = plsc.VectorSubcoreMesh(core_axis_name="c", subcore_axis_name="s")


def _deg_body(dst_hbm, pd_hbm, dstv, onesv, zerov, dacc):
    c = lax.axis_index("c")
    s = lax.axis_index("s")
    wid = c * 16 + s
    for i in range(CHUNK_D // 16):
        onesv[pl.ds(i * 16, 16)] = jnp.full((16,), 1.0, jnp.float32)
    for i in range(RPT_D // 16):
        zerov[pl.ds(i * 16, 16)] = jnp.zeros((16,), jnp.float32)
    pltpu.sync_copy(zerov, dacc.at[pl.ds(s * RPT_D, RPT_D)])
    plsc.subcore_barrier()

    def body(j, carry):
        off = (wid * CPT_D + j) * CHUNK_D
        pltpu.sync_copy(dst_hbm.at[pl.ds(off, CHUNK_D)], dstv)
        pltpu.sync_copy(onesv, dacc.at[dstv], add=True)
        return carry

    lax.fori_loop(0, CPT_D, body, 0)
    plsc.subcore_barrier()
    pltpu.sync_copy(dacc.at[pl.ds(s * RPT_D, RPT_D)],
                    pd_hbm.at[c, pl.ds(s * RPT_D, RPT_D)])


_deg_kernel = functools.partial(
    pl.kernel,
    out_type=jax.ShapeDtypeStruct((2, N_DEG), jnp.float32),
    mesh=_mesh,
    scratch_types=[
        pltpu.VMEM((CHUNK_D,), jnp.int32),
        pltpu.VMEM((CHUNK_D,), jnp.float32),
        pltpu.VMEM((RPT_D,), jnp.float32),
        pltpu.VMEM_SHARED((N_DEG,), jnp.float32),
    ],
)(_deg_body)


def _mp_body(src_hbm, dst_hbm, u0_hbm, u1_hbm, z_hbm, s0_hbm, s1_hbm,
             srcv0, srcv1, srcv2, srcv3,
             dstv0, dstv1, dstv2, dstv3,
             rows0, rows1, rows2,
             acc, zsem, isem, gsem, ssem):
    srcv = [srcv0, srcv1, srcv2, srcv3]
    dstv = [dstv0, dstv1, dstv2, dstv3]
    rows = [rows0, rows1, rows2]
    c = lax.axis_index("c")
    s = lax.axis_index("s")
    zcp = pltpu.async_copy(z_hbm.at[pl.ds(s * RPT, RPT)],
                           acc.at[pl.ds(s * RPT, RPT)], zsem)

    idx_cps = {}

    def load_idx(j):
        b = j % NIB
        off = (s * CPT + j) * CHUNK
        d1 = pltpu.async_copy(src_hbm.at[pl.ds(off, CHUNK)], srcv[b], isem)
        d2 = pltpu.async_copy(dst_hbm.at[pl.ds(off, CHUNK)], dstv[b], isem)
        idx_cps[j] = (d1, d2)

    def start_gather(j):
        ib = j % NIB
        b = j % NBUF

        @pl.when(c == 0)
        def _():
            pltpu.async_copy(u0_hbm.at[srcv[ib]], rows[b], gsem)

        @pl.when(c == 1)
        def _():
            pltpu.async_copy(u1_hbm.at[srcv[ib]], rows[b], gsem)

    def wait_gather(j):
        ib = j % NIB
        b = j % NBUF
        pltpu.make_async_copy(u0_hbm.at[srcv[ib]], rows[b], gsem).wait()

    def start_scatter(j):
        ib = j % NIB
        b = j % NBUF
        pltpu.async_copy(rows[b], acc.at[dstv[ib]], ssem, add=True)

    def wait_scatter(j):
        ib = j % NIB
        b = j % NBUF
        pltpu.make_async_copy(rows[b], acc.at[dstv[ib]], ssem).wait()

    load_idx(0)
    load_idx(1)
    idx_cps[0][0].wait()
    idx_cps[0][1].wait()
    start_gather(0)
    zcp.wait()
    plsc.subcore_barrier()

    for j in range(CPT):
        wait_gather(j)
        if j >= 2:
            wait_scatter(j - 2)
        start_scatter(j)
        if j + 2 < CPT:
            load_idx(j + 2)
        if j + 1 < CPT:
            idx_cps[j + 1][0].wait()
            idx_cps[j + 1][1].wait()
            start_gather(j + 1)
    wait_scatter(CPT - 2)
    wait_scatter(CPT - 1)
    plsc.subcore_barrier()

    @pl.when(c == 0)
    def _():
        pltpu.sync_copy(acc.at[pl.ds(s * RPT, RPT)],
                        s0_hbm.at[pl.ds(s * RPT, RPT)])

    @pl.when(c == 1)
    def _():
        pltpu.sync_copy(acc.at[pl.ds(s * RPT, RPT)],
                        s1_hbm.at[pl.ds(s * RPT, RPT)])


_mp_kernel = functools.partial(
    pl.kernel,
    out_type=(jax.ShapeDtypeStruct((N_PAD, DH), jnp.float32),
              jax.ShapeDtypeStruct((N_PAD, DH), jnp.float32)),
    mesh=_mesh,
    scratch_types=(
        [pltpu.VMEM((CHUNK,), jnp.int32)] * 8
        + [pltpu.VMEM((CHUNK, DH), jnp.float32)] * 3
    ) + [
        pltpu.VMEM_SHARED((N_PAD, DH), jnp.float32),
        pltpu.SemaphoreType.DMA,
        pltpu.SemaphoreType.DMA,
        pltpu.SemaphoreType.DMA,
        pltpu.SemaphoreType.DMA,
    ],
)(_mp_body)


def _enc_body(x_ref, dis_ref, We_ref, be_ref, W1_ref, b1_ref,
              h0_ref, u0_ref, u1_ref):
    h0 = jnp.dot(x_ref[...], We_ref[...],
                 preferred_element_type=jnp.float32) + be_ref[...]
    t1 = jnp.dot(h0, W1_ref[...],
                 preferred_element_type=jnp.float32) + b1_ref[...]
    u = t1 * dis_ref[...]
    h0_ref[...] = h0
    u0_ref[...] = u[:, :DH]
    u1_ref[...] = u[:, DH:]


_enc_call = pl.pallas_call(
    _enc_body,
    grid=(GRID,),
    in_specs=[
        pl.BlockSpec((BLK, D), lambda i: (i, 0)),
        pl.BlockSpec((BLK, 1), lambda i: (i, 0)),
        pl.BlockSpec((D, D), lambda i: (0, 0)),
        pl.BlockSpec((1, D), lambda i: (0, 0)),
        pl.BlockSpec((D, D), lambda i: (0, 0)),
        pl.BlockSpec((1, D), lambda i: (0, 0)),
    ],
    out_specs=[
        pl.BlockSpec((BLK, D), lambda i: (i, 0)),
        pl.BlockSpec((BLK, DH), lambda i: (i, 0)),
        pl.BlockSpec((BLK, DH), lambda i: (i, 0)),
    ],
    out_shape=[
        jax.ShapeDtypeStruct((N, D), jnp.float32),
        jax.ShapeDtypeStruct((N_PAD, DH), jnp.float32),
        jax.ShapeDtypeStruct((N_PAD, DH), jnp.float32),
    ],
)


def _mid_body(s0_ref, s1_ref, u0_ref, u1_ref, dis_ref, h_ref, W_ref, b_ref,
              hn_ref, v0_ref, v1_ref):
    pre = jnp.concatenate([s0_ref[...] + u0_ref[...],
                           s1_ref[...] + u1_ref[...]], axis=1)
    hn = jnp.maximum(pre * dis_ref[...], 0.0) + h_ref[...]
    t = jnp.dot(hn, W_ref[...], preferred_element_type=jnp.float32) + b_ref[...]
    u = t * dis_ref[...]
    hn_ref[...] = hn
    v0_ref[...] = u[:, :DH]
    v1_ref[...] = u[:, DH:]


_mid_call = pl.pallas_call(
    _mid_body,
    grid=(GRID,),
    in_specs=[
        pl.BlockSpec((BLK, DH), lambda i: (i, 0)),
        pl.BlockSpec((BLK, DH), lambda i: (i, 0)),
        pl.BlockSpec((BLK, DH), lambda i: (i, 0)),
        pl.BlockSpec((BLK, DH), lambda i: (i, 0)),
        pl.BlockSpec((BLK, 1), lambda i: (i, 0)),
        pl.BlockSpec((BLK, D), lambda i: (i, 0)),
        pl.BlockSpec((D, D), lambda i: (0, 0)),
        pl.BlockSpec((1, D), lambda i: (0, 0)),
    ],
    out_specs=[
        pl.BlockSpec((BLK, D), lambda i: (i, 0)),
        pl.BlockSpec((BLK, DH), lambda i: (i, 0)),
        pl.BlockSpec((BLK, DH), lambda i: (i, 0)),
    ],
    out_shape=[
        jax.ShapeDtypeStruct((N, D), jnp.float32),
        jax.ShapeDtypeStruct((N_PAD, DH), jnp.float32),
        jax.ShapeDtypeStruct((N_PAD, DH), jnp.float32),
    ],
)


def _dec_body(s0_ref, s1_ref, u0_ref, u1_ref, dis_ref, h_ref, W_ref, b_ref,
              out_ref):
    pre = jnp.concatenate([s0_ref[...] + u0_ref[...],
                           s1_ref[...] + u1_ref[...]], axis=1)
    hn = jnp.maximum(pre * dis_ref[...], 0.0) + h_ref[...]
    out_ref[...] = jnp.dot(hn, W_ref[...],
                           preferred_element_type=jnp.float32) + b_ref[...]


_dec_call = pl.pallas_call(
    _dec_body,
    grid=(GRID,),
    in_specs=[
        pl.BlockSpec((BLK, DH), lambda i: (i, 0)),
        pl.BlockSpec((BLK, DH), lambda i: (i, 0)),
        pl.BlockSpec((BLK, DH), lambda i: (i, 0)),
        pl.BlockSpec((BLK, DH), lambda i: (i, 0)),
        pl.BlockSpec((BLK, 1), lambda i: (i, 0)),
        pl.BlockSpec((BLK, D), lambda i: (i, 0)),
        pl.BlockSpec((D, D), lambda i: (0, 0)),
        pl.BlockSpec((1, D), lambda i: (0, 0)),
    ],
    out_specs=pl.BlockSpec((BLK, D), lambda i: (i, 0)),
    out_shape=jax.ShapeDtypeStruct((N, D), jnp.float32),
)


def kernel(x, edge_index, W_enc, b_enc, W_c1, b_c1, W_c2, b_c2, W_dec, b_dec):
    src = edge_index[0]
    dst = edge_index[1]
    pad = jnp.full((E_PAD - E,), N, jnp.int32)
    srcp = jnp.concatenate([src, pad])
    dstp = jnp.concatenate([dst, pad])

    pd = _deg_kernel(dstp)
    deg = pd[0, :N] + pd[1, :N] + 1.0
    disc = lax.rsqrt(deg).reshape(N, 1)

    be = b_enc.reshape(1, D)
    b1 = b_c1.reshape(1, D)
    b2 = b_c2.reshape(1, D)
    bd = b_dec.reshape(1, D)
    zeros_acc = jnp.zeros((N_PAD, DH), jnp.float32)

    h0, u0, u1 = _enc_call(x, disc, W_enc, be, W_c1, b1)
    s0, s1 = _mp_kernel(srcp, dstp, u0, u1, zeros_acc)
    h1, v0, v1 = _mid_call(s0, s1, u0, u1, disc, h0, W_c2, b2)
    r0, r1 = _mp_kernel(srcp, dstp, v0, v1, zeros_acc)
    out = _dec_call(r0, r1, v0, v1, disc, h1, W_dec, bd)
    return out


# speedup vs baseline: 1.0435x; 1.0112x over previous
import functools

import jax
import jax.numpy as jnp
from jax import lax
from jax.experimental import pallas as pl
from jax.experimental.pallas import tpu as pltpu
from jax.experimental.pallas import tpu_sc as plsc

N = 10000
D = 256
DH = 128
E = 160000

N_PAD = 10112
RPT = N_PAD // 16

CHUNK = 120
CPT = 84
E_PAD = CHUNK * 16 * CPT
NBUF = 3
NIB = 4

N_DEG = 10240
RPT_D = N_DEG // 16
CHUNK_D = 1680
CPT_D = (E_PAD // CHUNK_D) // 32

BLK = 2048
GRID = 5

_mesh = plsc.VectorSubcoreMesh(core_axis_name="c", subcore_axis_name="s")


def _deg_body(dst_hbm, pd_hbm, dstv, onesv, zerov, dacc):
    c = lax.axis_index("c")
    s = lax.axis_index("s")
    wid = c * 16 + s
    for i in range(CHUNK_D // 16):
        onesv[pl.ds(i * 16, 16)] = jnp.full((16,), 1.0, jnp.float32)
    for i in range(RPT_D // 16):
        zerov[pl.ds(i * 16, 16)] = jnp.zeros((16,), jnp.float32)
    pltpu.sync_copy(zerov, dacc.at[pl.ds(s * RPT_D, RPT_D)])
    plsc.subcore_barrier()

    def body(j, carry):
        off = (wid * CPT_D + j) * CHUNK_D
        pltpu.sync_copy(dst_hbm.at[pl.ds(off, CHUNK_D)], dstv)
        pltpu.sync_copy(onesv, dacc.at[dstv], add=True)
        return carry

    lax.fori_loop(0, CPT_D, body, 0)
    plsc.subcore_barrier()
    pltpu.sync_copy(dacc.at[pl.ds(s * RPT_D, RPT_D)],
                    pd_hbm.at[c, pl.ds(s * RPT_D, RPT_D)])


_deg_kernel = functools.partial(
    pl.kernel,
    out_type=jax.ShapeDtypeStruct((2, N_DEG), jnp.float32),
    mesh=_mesh,
    scratch_types=[
        pltpu.VMEM((CHUNK_D,), jnp.int32),
        pltpu.VMEM((CHUNK_D,), jnp.float32),
        pltpu.VMEM((RPT_D,), jnp.float32),
        pltpu.VMEM_SHARED((N_DEG,), jnp.float32),
    ],
)(_deg_body)


def _mp_body(src_hbm, dst_hbm, u0_hbm, u1_hbm, z_hbm, s0_hbm, s1_hbm,
             srcv0, srcv1, srcv2, srcv3,
             dstv0, dstv1, dstv2, dstv3,
             rows0, rows1, rows2,
             acc, zsem, isem, gsem, ssem):
    srcv = [srcv0, srcv1, srcv2, srcv3]
    dstv = [dstv0, dstv1, dstv2, dstv3]
    rows = [rows0, rows1, rows2]
    c = lax.axis_index("c")
    s = lax.axis_index("s")
    zcp = pltpu.async_copy(z_hbm.at[pl.ds(s * RPT, RPT)],
                           acc.at[pl.ds(s * RPT, RPT)], zsem)

    idx_cps = {}

    def load_idx(j):
        b = j % NIB
        off = (s * CPT + j) * CHUNK
        d1 = pltpu.async_copy(src_hbm.at[pl.ds(off, CHUNK)], srcv[b], isem)
        d2 = pltpu.async_copy(dst_hbm.at[pl.ds(off, CHUNK)], dstv[b], isem)
        idx_cps[j] = (d1, d2)

    def start_gather(j):
        ib = j % NIB
        b = j % NBUF

        @pl.when(c == 0)
        def _():
            pltpu.async_copy(u0_hbm.at[srcv[ib]], rows[b], gsem)

        @pl.when(c == 1)
        def _():
            pltpu.async_copy(u1_hbm.at[srcv[ib]], rows[b], gsem)

    def wait_gather(j):
        ib = j % NIB
        b = j % NBUF
        pltpu.make_async_copy(u0_hbm.at[srcv[ib]], rows[b], gsem).wait()

    def start_scatter(j):
        ib = j % NIB
        b = j % NBUF
        pltpu.async_copy(rows[b], acc.at[dstv[ib]], ssem, add=True)

    def wait_scatter(j):
        ib = j % NIB
        b = j % NBUF
        pltpu.make_async_copy(rows[b], acc.at[dstv[ib]], ssem).wait()

    load_idx(0)
    load_idx(1)
    idx_cps[0][0].wait()
    idx_cps[0][1].wait()
    start_gather(0)
    zcp.wait()
    plsc.subcore_barrier()

    for j in range(CPT):
        wait_gather(j)
        if j >= 2:
            wait_scatter(j - 2)
        start_scatter(j)
        if j + 2 < CPT:
            load_idx(j + 2)
        if j + 1 < CPT:
            idx_cps[j + 1][0].wait()
            idx_cps[j + 1][1].wait()
            start_gather(j + 1)
    wait_scatter(CPT - 2)
    wait_scatter(CPT - 1)
    plsc.subcore_barrier()

    @pl.when(c == 0)
    def _():
        pltpu.sync_copy(acc.at[pl.ds(s * RPT, RPT)],
                        s0_hbm.at[pl.ds(s * RPT, RPT)])

    @pl.when(c == 1)
    def _():
        pltpu.sync_copy(acc.at[pl.ds(s * RPT, RPT)],
                        s1_hbm.at[pl.ds(s * RPT, RPT)])


_mp_kernel = functools.partial(
    pl.kernel,
    out_type=(jax.ShapeDtypeStruct((N_PAD, DH), jnp.float32),
              jax.ShapeDtypeStruct((N_PAD, DH), jnp.float32)),
    mesh=_mesh,
    scratch_types=(
        [pltpu.VMEM((CHUNK,), jnp.int32)] * 8
        + [pltpu.VMEM((CHUNK, DH), jnp.float32)] * 3
    ) + [
        pltpu.VMEM_SHARED((N_PAD, DH), jnp.float32),
        pltpu.SemaphoreType.DMA,
        pltpu.SemaphoreType.DMA,
        pltpu.SemaphoreType.DMA,
        pltpu.SemaphoreType.DMA,
    ],
)(_mp_body)


def _enc_body(x_ref, dis_ref, We_ref, be_ref, W1_ref, b1_ref,
              h0_ref, u0_ref, u1_ref):
    h0 = jnp.dot(x_ref[...], We_ref[...],
                 preferred_element_type=jnp.float32) + be_ref[...]
    t1 = jnp.dot(h0, W1_ref[...],
                 preferred_element_type=jnp.float32) + b1_ref[...]
    u = t1 * dis_ref[...]
    h0_ref[...] = h0
    u0_ref[...] = u[:, :DH]
    u1_ref[...] = u[:, DH:]


_enc_call = pl.pallas_call(
    _enc_body,
    grid=(GRID,),
    in_specs=[
        pl.BlockSpec((BLK, D), lambda i: (i, 0)),
        pl.BlockSpec((BLK, 1), lambda i: (i, 0)),
        pl.BlockSpec((D, D), lambda i: (0, 0)),
        pl.BlockSpec((1, D), lambda i: (0, 0)),
        pl.BlockSpec((D, D), lambda i: (0, 0)),
        pl.BlockSpec((1, D), lambda i: (0, 0)),
    ],
    out_specs=[
        pl.BlockSpec((BLK, D), lambda i: (i, 0)),
        pl.BlockSpec((BLK, DH), lambda i: (i, 0)),
        pl.BlockSpec((BLK, DH), lambda i: (i, 0)),
    ],
    out_shape=[
        jax.ShapeDtypeStruct((N, D), jnp.float32),
        jax.ShapeDtypeStruct((N_PAD, DH), jnp.float32),
        jax.ShapeDtypeStruct((N_PAD, DH), jnp.float32),
    ],
)


def _mid_body(s0_ref, s1_ref, u0_ref, u1_ref, dis_ref, h_ref, W_ref, b_ref,
              hn_ref, v0_ref, v1_ref):
    pre = jnp.concatenate([s0_ref[...] + u0_ref[...],
                           s1_ref[...] + u1_ref[...]], axis=1)
    hn = jnp.maximum(pre * dis_ref[...], 0.0) + h_ref[...]
    t = jnp.dot(hn, W_ref[...], preferred_element_type=jnp.float32) + b_ref[...]
    u = t * dis_ref[...]
    hn_ref[...] = hn
    v0_ref[...] = u[:, :DH]
    v1_ref[...] = u[:, DH:]


_mid_call = pl.pallas_call(
    _mid_body,
    grid=(GRID,),
    in_specs=[
        pl.BlockSpec((BLK, DH), lambda i: (i, 0)),
        pl.BlockSpec((BLK, DH), lambda i: (i, 0)),
        pl.BlockSpec((BLK, DH), lambda i: (i, 0)),
        pl.BlockSpec((BLK, DH), lambda i: (i, 0)),
        pl.BlockSpec((BLK, 1), lambda i: (i, 0)),
        pl.BlockSpec((BLK, D), lambda i: (i, 0)),
        pl.BlockSpec((D, D), lambda i: (0, 0)),
        pl.BlockSpec((1, D), lambda i: (0, 0)),
    ],
    out_specs=[
        pl.BlockSpec((BLK, D), lambda i: (i, 0)),
        pl.BlockSpec((BLK, DH), lambda i: (i, 0)),
        pl.BlockSpec((BLK, DH), lambda i: (i, 0)),
    ],
    out_shape=[
        jax.ShapeDtypeStruct((N, D), jnp.float32),
        jax.ShapeDtypeStruct((N_PAD, DH), jnp.float32),
        jax.ShapeDtypeStruct((N_PAD, DH), jnp.float32),
    ],
)


def _dec_body(s0_ref, s1_ref, u0_ref, u1_ref, dis_ref, h_ref, W_ref, b_ref,
              out_ref):
    pre = jnp.concatenate([s0_ref[...] + u0_ref[...],
                           s1_ref[...] + u1_ref[...]], axis=1)
    hn = jnp.maximum(pre * dis_ref[...], 0.0) + h_ref[...]
    out_ref[...] = jnp.dot(hn, W_ref[...],
                           preferred_element_type=jnp.float32) + b_ref[...]


_dec_call = pl.pallas_call(
    _dec_body,
    grid=(GRID,),
    in_specs=[
        pl.BlockSpec((BLK, DH), lambda i: (i, 0)),
        pl.BlockSpec((BLK, DH), lambda i: (i, 0)),
        pl.BlockSpec((BLK, DH), lambda i: (i, 0)),
        pl.BlockSpec((BLK, DH), lambda i: (i, 0)),
        pl.BlockSpec((BLK, 1), lambda i: (i, 0)),
        pl.BlockSpec((BLK, D), lambda i: (i, 0)),
        pl.BlockSpec((D, D), lambda i: (0, 0)),
        pl.BlockSpec((1, D), lambda i: (0, 0)),
    ],
    out_specs=pl.BlockSpec((BLK, D), lambda i: (i, 0)),
    out_shape=jax.ShapeDtypeStruct((N, D), jnp.float32),
)


def kernel(x, edge_index, W_enc, b_enc, W_c1, b_c1, W_c2, b_c2, W_dec, b_dec):
    src = edge_index[0]
    dst = edge_index[1]
    pad = jnp.full((E_PAD - E,), N, jnp.int32)
    srcp = jnp.concatenate([src, pad])
    dstp = jnp.concatenate([dst, pad])

    pd = _deg_kernel(dstp)
    deg = pd[0, :N] + pd[1, :N] + 1.0
    disc = lax.rsqrt(deg).reshape(N, 1)

    be = b_enc.reshape(1, D)
    b1 = b_c1.reshape(1, D)
    b2 = b_c2.reshape(1, D)
    bd = b_dec.reshape(1, D)
    zeros_acc = jnp.zeros((N_PAD, DH), jnp.float32)

    h0, u0, u1 = _enc_call(x, disc, W_enc, be, W_c1, b1)
    s0, s1 = _mp_kernel(srcp, dstp, u0, u1, zeros_acc)
    h1, v0, v1 = _mid_call(s0, s1, u0, u1, disc, h0, W_c2, b2)
    r0, r1 = _mp_kernel(srcp, dstp, v0, v1, zeros_acc)
    out = _dec_call(r0, r1, v0, v1, disc, h1, W_dec, bd)
    return out


# speedup vs baseline: 1.0478x; 1.0041x over previous
import functools

import jax
import jax.numpy as jnp
from jax import lax
from jax.experimental import pallas as pl
from jax.experimental.pallas import tpu as pltpu
from jax.experimental.pallas import tpu_sc as plsc

N = 10000
D = 256
DH = 128
E = 160000

N_PAD = 10112
RPT = N_PAD // 16

CHUNK = 120
CPT = 84
E_PAD = CHUNK * 16 * CPT
NBUF = 3
NIB = 4

N_DEG = 10240
RPT_D = N_DEG // 16
CHUNK_D = 1680
CPT_D = (E_PAD // CHUNK_D) // 32

BLK = 2560
GRID = 4

_mesh = plsc.VectorSubcoreMesh(core_axis_name="c", subcore_axis_name="s")


def _deg_body(dst_hbm, pd_hbm, dstv, onesv, zerov, dacc):
    c = lax.axis_index("c")
    s = lax.axis_index("s")
    wid = c * 16 + s
    for i in range(CHUNK_D // 16):
        onesv[pl.ds(i * 16, 16)] = jnp.full((16,), 1.0, jnp.float32)
    for i in range(RPT_D // 16):
        zerov[pl.ds(i * 16, 16)] = jnp.zeros((16,), jnp.float32)
    pltpu.sync_copy(zerov, dacc.at[pl.ds(s * RPT_D, RPT_D)])
    plsc.subcore_barrier()

    def body(j, carry):
        off = (wid * CPT_D + j) * CHUNK_D
        pltpu.sync_copy(dst_hbm.at[pl.ds(off, CHUNK_D)], dstv)
        pltpu.sync_copy(onesv, dacc.at[dstv], add=True)
        return carry

    lax.fori_loop(0, CPT_D, body, 0)
    plsc.subcore_barrier()
    pltpu.sync_copy(dacc.at[pl.ds(s * RPT_D, RPT_D)],
                    pd_hbm.at[c, pl.ds(s * RPT_D, RPT_D)])


_deg_kernel = functools.partial(
    pl.kernel,
    out_type=jax.ShapeDtypeStruct((2, N_DEG), jnp.float32),
    mesh=_mesh,
    scratch_types=[
        pltpu.VMEM((CHUNK_D,), jnp.int32),
        pltpu.VMEM((CHUNK_D,), jnp.float32),
        pltpu.VMEM((RPT_D,), jnp.float32),
        pltpu.VMEM_SHARED((N_DEG,), jnp.float32),
    ],
)(_deg_body)


def _mp_body(src_hbm, dst_hbm, u0_hbm, u1_hbm, z_hbm, s0_hbm, s1_hbm,
             srcv0, srcv1, srcv2, srcv3,
             dstv0, dstv1, dstv2, dstv3,
             rows0, rows1, rows2,
             acc, zsem, isem, gsem, ssem):
    srcv = [srcv0, srcv1, srcv2, srcv3]
    dstv = [dstv0, dstv1, dstv2, dstv3]
    rows = [rows0, rows1, rows2]
    c = lax.axis_index("c")
    s = lax.axis_index("s")
    zcp = pltpu.async_copy(z_hbm.at[pl.ds(s * RPT, RPT)],
                           acc.at[pl.ds(s * RPT, RPT)], zsem)

    idx_cps = {}

    def load_idx(j):
        b = j % NIB
        off = (s * CPT + j) * CHUNK
        d1 = pltpu.async_copy(src_hbm.at[pl.ds(off, CHUNK)], srcv[b], isem)
        d2 = pltpu.async_copy(dst_hbm.at[pl.ds(off, CHUNK)], dstv[b], isem)
        idx_cps[j] = (d1, d2)

    def start_gather(j):
        ib = j % NIB
        b = j % NBUF

        @pl.when(c == 0)
        def _():
            pltpu.async_copy(u0_hbm.at[srcv[ib]], rows[b], gsem)

        @pl.when(c == 1)
        def _():
            pltpu.async_copy(u1_hbm.at[srcv[ib]], rows[b], gsem)

    def wait_gather(j):
        ib = j % NIB
        b = j % NBUF
        pltpu.make_async_copy(u0_hbm.at[srcv[ib]], rows[b], gsem).wait()

    def start_scatter(j):
        ib = j % NIB
        b = j % NBUF
        pltpu.async_copy(rows[b], acc.at[dstv[ib]], ssem, add=True)

    def wait_scatter(j):
        ib = j % NIB
        b = j % NBUF
        pltpu.make_async_copy(rows[b], acc.at[dstv[ib]], ssem).wait()

    load_idx(0)
    load_idx(1)
    idx_cps[0][0].wait()
    idx_cps[0][1].wait()
    start_gather(0)
    zcp.wait()
    plsc.subcore_barrier()

    for j in range(CPT):
        wait_gather(j)
        if j >= 2:
            wait_scatter(j - 2)
        start_scatter(j)
        if j + 2 < CPT:
            load_idx(j + 2)
        if j + 1 < CPT:
            idx_cps[j + 1][0].wait()
            idx_cps[j + 1][1].wait()
            start_gather(j + 1)
    wait_scatter(CPT - 2)
    wait_scatter(CPT - 1)
    plsc.subcore_barrier()

    @pl.when(c == 0)
    def _():
        pltpu.sync_copy(acc.at[pl.ds(s * RPT, RPT)],
                        s0_hbm.at[pl.ds(s * RPT, RPT)])

    @pl.when(c == 1)
    def _():
        pltpu.sync_copy(acc.at[pl.ds(s * RPT, RPT)],
                        s1_hbm.at[pl.ds(s * RPT, RPT)])


_mp_kernel = functools.partial(
    pl.kernel,
    out_type=(jax.ShapeDtypeStruct((N_PAD, DH), jnp.float32),
              jax.ShapeDtypeStruct((N_PAD, DH), jnp.float32)),
    mesh=_mesh,
    scratch_types=(
        [pltpu.VMEM((CHUNK,), jnp.int32)] * 8
        + [pltpu.VMEM((CHUNK, DH), jnp.float32)] * 3
    ) + [
        pltpu.VMEM_SHARED((N_PAD, DH), jnp.float32),
        pltpu.SemaphoreType.DMA,
        pltpu.SemaphoreType.DMA,
        pltpu.SemaphoreType.DMA,
        pltpu.SemaphoreType.DMA,
    ],
)(_mp_body)


def _enc_body(x_ref, dis_ref, We_ref, be_ref, W1_ref, b1_ref,
              h0_ref, u0_ref, u1_ref):
    h0 = jnp.dot(x_ref[...], We_ref[...],
                 preferred_element_type=jnp.float32) + be_ref[...]
    t1 = jnp.dot(h0, W1_ref[...],
                 preferred_element_type=jnp.float32) + b1_ref[...]
    u = t1 * dis_ref[...]
    h0_ref[...] = h0
    u0_ref[...] = u[:, :DH]
    u1_ref[...] = u[:, DH:]


_enc_call = pl.pallas_call(
    _enc_body,
    grid=(GRID,),
    in_specs=[
        pl.BlockSpec((BLK, D), lambda i: (i, 0)),
        pl.BlockSpec((BLK, 1), lambda i: (i, 0)),
        pl.BlockSpec((D, D), lambda i: (0, 0)),
        pl.BlockSpec((1, D), lambda i: (0, 0)),
        pl.BlockSpec((D, D), lambda i: (0, 0)),
        pl.BlockSpec((1, D), lambda i: (0, 0)),
    ],
    out_specs=[
        pl.BlockSpec((BLK, D), lambda i: (i, 0)),
        pl.BlockSpec((BLK, DH), lambda i: (i, 0)),
        pl.BlockSpec((BLK, DH), lambda i: (i, 0)),
    ],
    out_shape=[
        jax.ShapeDtypeStruct((N, D), jnp.float32),
        jax.ShapeDtypeStruct((N_PAD, DH), jnp.float32),
        jax.ShapeDtypeStruct((N_PAD, DH), jnp.float32),
    ],
)


def _mid_body(s0_ref, s1_ref, u0_ref, u1_ref, dis_ref, h_ref, W_ref, b_ref,
              hn_ref, v0_ref, v1_ref):
    pre = jnp.concatenate([s0_ref[...] + u0_ref[...],
                           s1_ref[...] + u1_ref[...]], axis=1)
    hn = jnp.maximum(pre * dis_ref[...], 0.0) + h_ref[...]
    t = jnp.dot(hn, W_ref[...], preferred_element_type=jnp.float32) + b_ref[...]
    u = t * dis_ref[...]
    hn_ref[...] = hn
    v0_ref[...] = u[:, :DH]
    v1_ref[...] = u[:, DH:]


_mid_call = pl.pallas_call(
    _mid_body,
    grid=(GRID,),
    in_specs=[
        pl.BlockSpec((BLK, DH), lambda i: (i, 0)),
        pl.BlockSpec((BLK, DH), lambda i: (i, 0)),
        pl.BlockSpec((BLK, DH), lambda i: (i, 0)),
        pl.BlockSpec((BLK, DH), lambda i: (i, 0)),
        pl.BlockSpec((BLK, 1), lambda i: (i, 0)),
        pl.BlockSpec((BLK, D), lambda i: (i, 0)),
        pl.BlockSpec((D, D), lambda i: (0, 0)),
        pl.BlockSpec((1, D), lambda i: (0, 0)),
    ],
    out_specs=[
        pl.BlockSpec((BLK, D), lambda i: (i, 0)),
        pl.BlockSpec((BLK, DH), lambda i: (i, 0)),
        pl.BlockSpec((BLK, DH), lambda i: (i, 0)),
    ],
    out_shape=[
        jax.ShapeDtypeStruct((N, D), jnp.float32),
        jax.ShapeDtypeStruct((N_PAD, DH), jnp.float32),
        jax.ShapeDtypeStruct((N_PAD, DH), jnp.float32),
    ],
)


def _dec_body(s0_ref, s1_ref, u0_ref, u1_ref, dis_ref, h_ref, W_ref, b_ref,
              out_ref):
    pre = jnp.concatenate([s0_ref[...] + u0_ref[...],
                           s1_ref[...] + u1_ref[...]], axis=1)
    hn = jnp.maximum(pre * dis_ref[...], 0.0) + h_ref[...]
    out_ref[...] = jnp.dot(hn, W_ref[...],
                           preferred_element_type=jnp.float32) + b_ref[...]


_dec_call = pl.pallas_call(
    _dec_body,
    grid=(GRID,),
    in_specs=[
        pl.BlockSpec((BLK, DH), lambda i: (i, 0)),
        pl.BlockSpec((BLK, DH), lambda i: (i, 0)),
        pl.BlockSpec((BLK, DH), lambda i: (i, 0)),
        pl.BlockSpec((BLK, DH), lambda i: (i, 0)),
        pl.BlockSpec((BLK, 1), lambda i: (i, 0)),
        pl.BlockSpec((BLK, D), lambda i: (i, 0)),
        pl.BlockSpec((D, D), lambda i: (0, 0)),
        pl.BlockSpec((1, D), lambda i: (0, 0)),
    ],
    out_specs=pl.BlockSpec((BLK, D), lambda i: (i, 0)),
    out_shape=jax.ShapeDtypeStruct((N, D), jnp.float32),
)


def kernel(x, edge_index, W_enc, b_enc, W_c1, b_c1, W_c2, b_c2, W_dec, b_dec):
    src = edge_index[0]
    dst = edge_index[1]
    pad = jnp.full((E_PAD - E,), N, jnp.int32)
    srcp = jnp.concatenate([src, pad])
    dstp = jnp.concatenate([dst, pad])

    pd = _deg_kernel(dstp)
    deg = pd[0, :N] + pd[1, :N] + 1.0
    disc = lax.rsqrt(deg).reshape(N, 1)

    be = b_enc.reshape(1, D)
    b1 = b_c1.reshape(1, D)
    b2 = b_c2.reshape(1, D)
    bd = b_dec.reshape(1, D)
    zeros_acc = jnp.zeros((N_PAD, DH), jnp.float32)

    h0, u0, u1 = _enc_call(x, disc, W_enc, be, W_c1, b1)
    s0, s1 = _mp_kernel(srcp, dstp, u0, u1, zeros_acc)
    h1, v0, v1 = _mid_call(s0, s1, u0, u1, disc, h0, W_c2, b2)
    r0, r1 = _mp_kernel(srcp, dstp, v0, v1, zeros_acc)
    out = _dec_call(r0, r1, v0, v1, disc, h1, W_dec, bd)
    return out
